# Initial kernel scaffold; baseline (speedup 1.0000x reference)
import jax
import jax.numpy as jnp
from jax.experimental import pallas as pl


def kernel(sequence_tensor, positions):
    raise NotImplementedError("write your pallas kernel here")


# speedup vs baseline: 1.4013x; 1.4013x over previous
import functools

import jax
import jax.numpy as jnp
from jax import lax
from jax.experimental import pallas as pl
from jax.experimental.pallas import tpu as pltpu
from jax.experimental.pallas import tpu_sc as plsc


def kernel(sequence_tensor, positions):
    batch_size, seq_length, width = sequence_tensor.shape
    nb, npos = positions.shape
    total = nb * npos

    flat_table = sequence_tensor.reshape(batch_size * seq_length, width)
    pos32 = positions.astype(jnp.int32).reshape(-1)

    info = plsc.get_sparse_core_info()
    num_cores = info.num_cores
    num_workers = num_cores * info.num_subcores
    b_per_w = total // num_workers

    mesh = plsc.VectorSubcoreMesh(core_axis_name="c", subcore_axis_name="s")

    @functools.partial(
        pl.kernel,
        mesh=mesh,
        out_type=jax.ShapeDtypeStruct((total, width), jnp.float32),
        scratch_types=[
            pltpu.VMEM((b_per_w,), jnp.int32),
            pltpu.VMEM((b_per_w, width), jnp.float32),
            pltpu.SemaphoreType.DMA,
        ],
    )
    def gather_k(table_hbm, idx_hbm, out_hbm, idx_v, rows_v, sem):
        wid = lax.axis_index("s") * num_cores + lax.axis_index("c")
        base = wid * b_per_w
        pltpu.sync_copy(idx_hbm.at[pl.ds(base, b_per_w)], idx_v)
        off = (base // npos) * seq_length
        for i in range(b_per_w // 16):
            sl = pl.ds(i * 16, 16)
            idx_v[sl] = idx_v[sl] + off
        pltpu.async_copy(table_hbm.at[idx_v], rows_v, sem).wait()
        pltpu.sync_copy(rows_v, out_hbm.at[pl.ds(base, b_per_w)])

    return gather_k(flat_table, pos32)
